# baseline (speedup 1.0000x reference)
import functools

import jax
import jax.numpy as jnp
from jax import lax
from jax.experimental import pallas as pl
from jax.experimental.pallas import tpu as pltpu
from jax.experimental.pallas import tpu_sc as plsc

NUM_TICKERS = 1000
EMBED_DIM = 32
LANES = 128
BATCH = 16384

_INFO = plsc.get_sparse_core_info()
_NC = _INFO.num_cores
_NS = _INFO.num_subcores
_NW = _NC * _NS
_B_PER_W = BATCH // _NW
_NCHUNK = 4
_CHUNK = _B_PER_W // _NCHUNK

_MESH = plsc.VectorSubcoreMesh(core_axis_name="c", subcore_axis_name="s")


@functools.partial(
    pl.kernel,
    mesh=_MESH,
    out_type=jax.ShapeDtypeStruct((BATCH, LANES), jnp.float32),
    scratch_types=[
        pltpu.VMEM((_NCHUNK, _CHUNK), jnp.int32),
        pltpu.VMEM((_B_PER_W, EMBED_DIM), jnp.float32),
        [pltpu.SemaphoreType.DMA] * _NCHUNK,
        [pltpu.SemaphoreType.DMA] * _NCHUNK,
    ],
    compiler_params=pltpu.CompilerParams(use_tc_tiling_on_sc=False),
)
def _embed_gather(tickers_hbm, table_hbm, out_hbm, idx_v, rows_v, gsems, wsems):
    wid = lax.axis_index("s") * _NC + lax.axis_index("c")
    base = wid * _B_PER_W
    pltpu.sync_copy(tickers_hbm.at[pl.ds(wid * _NCHUNK, _NCHUNK)], idx_v)
    gathers = []
    for j in range(_NCHUNK):
        gathers.append(pltpu.async_copy(
            table_hbm.at[idx_v.at[j]],
            rows_v.at[pl.ds(j * _CHUNK, _CHUNK)],
            gsems[j],
        ))
    writes = []
    for j in range(_NCHUNK):
        gathers[j].wait()
        writes.append(pltpu.async_copy(
            rows_v.at[pl.ds(j * _CHUNK, _CHUNK)],
            out_hbm.at[pl.ds(base + j * _CHUNK, _CHUNK), pl.ds(0, EMBED_DIM)],
            wsems[j],
        ))
    for w in writes:
        w.wait()


def kernel(tickers, table):
    t128 = tickers.astype(jnp.int32).reshape(BATCH // LANES, LANES)
    padded = _embed_gather(t128, table)
    return padded[:, :EMBED_DIM]
